# baseline (speedup 1.0000x reference)
import functools

import jax
import jax.numpy as jnp
from jax import lax
from jax.experimental import pallas as pl
from jax.experimental.pallas import tpu as pltpu
from jax.experimental.pallas import tpu_sc as plsc

_ROWS = 100000
_COLS = 512
_NW = 32
_WROWS = 3120
_BLOCK = 120
_NBLK = _WROWS // _BLOCK
_MAIN = _NW * _WROWS
_TAIL = _ROWS - _MAIN


@functools.partial(
    pl.kernel,
    out_type=jax.ShapeDtypeStruct((_ROWS, _COLS), jnp.float32),
    mesh=plsc.VectorSubcoreMesh(core_axis_name="c", subcore_axis_name="s"),
    scratch_types=[
        pltpu.VMEM((_BLOCK, _COLS), jnp.float32),
        pltpu.VMEM((_BLOCK, _COLS), jnp.float32),
        pltpu.SemaphoreType.DMA((2,)),
        pltpu.SemaphoreType.DMA((2,)),
    ],
)
def _sc_copy_assign(x_hbm, o_hbm, buf0, buf1, in_sems, out_sems):
    wid = lax.axis_index("s") * 2 + lax.axis_index("c")
    row0 = wid * _WROWS
    bufs = (buf0, buf1)

    def in_copy(k, b):
        return pltpu.make_async_copy(
            x_hbm.at[pl.ds(row0 + k * _BLOCK, _BLOCK)], bufs[b], in_sems.at[b])

    def out_copy(k, b):
        return pltpu.make_async_copy(
            bufs[b], o_hbm.at[pl.ds(row0 + k * _BLOCK, _BLOCK)], out_sems.at[b])

    in_copy(0, 0).start()
    for k in range(_NBLK):
        b = k % 2
        in_copy(k, b).wait()
        if k == 0:
            @pl.when(wid == 0)
            def _():
                ones = jnp.full((16,), 1.0, dtype=jnp.float32)
                for r in (1, 2):
                    for j in range(_COLS // 16):
                        bufs[0][r, pl.ds(j * 16, 16)] = ones
        out_copy(k, b).start()
        if k + 1 < _NBLK:
            nb = (k + 1) % 2
            if k >= 1:
                out_copy(k - 1, nb).wait()
            in_copy(k + 1, nb).start()
    out_copy(_NBLK - 2, (_NBLK - 2) % 2).wait()
    out_copy(_NBLK - 1, (_NBLK - 1) % 2).wait()

    @pl.when(wid < _TAIL // 8)
    def _():
        tstart = _MAIN + wid * 8
        pltpu.sync_copy(x_hbm.at[pl.ds(tstart, 8)], buf0.at[pl.ds(0, 8)])
        pltpu.sync_copy(buf0.at[pl.ds(0, 8)], o_hbm.at[pl.ds(tstart, 8)])


def kernel(x):
    return _sc_copy_assign(x)


# speedup vs baseline: 1.0057x; 1.0057x over previous
import functools

import jax
import jax.numpy as jnp
from jax import lax
from jax.experimental import pallas as pl
from jax.experimental.pallas import tpu as pltpu
from jax.experimental.pallas import tpu_sc as plsc

_ROWS = 100000
_COLS = 512
_NW = 32
_WROWS = 3120
_BLOCK = 80
_NBUF = 3
_NBLK = _WROWS // _BLOCK
_MAIN = _NW * _WROWS
_TAIL = _ROWS - _MAIN


@functools.partial(
    pl.kernel,
    out_type=jax.ShapeDtypeStruct((_ROWS, _COLS), jnp.float32),
    mesh=plsc.VectorSubcoreMesh(core_axis_name="c", subcore_axis_name="s"),
    scratch_types=[
        pltpu.VMEM((_NBUF, _BLOCK, _COLS), jnp.float32),
        pltpu.SemaphoreType.DMA((_NBUF,)),
        pltpu.SemaphoreType.DMA((_NBUF,)),
    ],
)
def _sc_copy_assign(x_hbm, o_hbm, bufs, in_sems, out_sems):
    wid = lax.axis_index("s") * 2 + lax.axis_index("c")
    row0 = wid * _WROWS

    def in_copy(k):
        b = k % _NBUF
        return pltpu.make_async_copy(
            x_hbm.at[pl.ds(row0 + k * _BLOCK, _BLOCK)], bufs.at[b], in_sems.at[b])

    def out_copy(k):
        b = k % _NBUF
        return pltpu.make_async_copy(
            bufs.at[b], o_hbm.at[pl.ds(row0 + k * _BLOCK, _BLOCK)], out_sems.at[b])

    for k in range(_NBUF - 1):
        in_copy(k).start()
    for k in range(_NBLK):
        in_copy(k).wait()
        if k == 0:
            @pl.when(wid == 0)
            def _():
                ones = jnp.full((16,), 1.0, dtype=jnp.float32)
                for r in (1, 2):
                    for j in range(_COLS // 16):
                        bufs[0, r, pl.ds(j * 16, 16)] = ones
        out_copy(k).start()
        if k + _NBUF - 1 < _NBLK:
            if k >= 1:
                out_copy(k - 1).wait()
            in_copy(k + _NBUF - 1).start()
    for k in range(_NBLK - _NBUF, _NBLK):
        out_copy(k).wait()

    @pl.when(wid < _TAIL // 8)
    def _():
        tstart = _MAIN + wid * 8
        pltpu.sync_copy(x_hbm.at[pl.ds(tstart, 8)], bufs.at[0].at[pl.ds(0, 8)])
        pltpu.sync_copy(bufs.at[0].at[pl.ds(0, 8)], o_hbm.at[pl.ds(tstart, 8)])


def kernel(x):
    return _sc_copy_assign(x)


# speedup vs baseline: 1.3019x; 1.2946x over previous
import jax
import jax.numpy as jnp
from jax.experimental import pallas as pl
from jax.experimental.pallas import tpu as pltpu

_ROWS = 100000
_COLS = 512
_BLOCK_ROWS = 8000


def _copy_assign_block(x_ref, o_ref):
    o_ref[...] = x_ref[...]

    @pl.when(pl.program_id(0) == 0)
    def _():
        o_ref[1:3, :] = jnp.ones((2, _COLS), dtype=o_ref.dtype)


def kernel(x):
    grid = (_ROWS + _BLOCK_ROWS - 1) // _BLOCK_ROWS
    return pl.pallas_call(
        _copy_assign_block,
        grid=(grid,),
        in_specs=[pl.BlockSpec((_BLOCK_ROWS, _COLS), lambda i: (i, 0))],
        out_specs=pl.BlockSpec((_BLOCK_ROWS, _COLS), lambda i: (i, 0)),
        out_shape=jax.ShapeDtypeStruct((_ROWS, _COLS), x.dtype),
        compiler_params=pltpu.CompilerParams(vmem_limit_bytes=64 * 1024 * 1024),
    )(x)
